# baseline (speedup 1.0000x reference)
import functools

import jax
import jax.numpy as jnp
from jax import lax
from jax.experimental import pallas as pl
from jax.experimental.pallas import tpu as pltpu
from jax.experimental.pallas import tpu_sc as plsc

_TOT = 32 * 512 * 512
_COLS = 512
_ROWS = _TOT // _COLS

_SC_NW = 32
_SC_CHUNK_R = 32
_SC_NCH = 4
_SC_ROWS = _SC_NW * _SC_NCH * _SC_CHUNK_R
_SC_BASE_ROW = _ROWS - _SC_ROWS
_SC_BASE = _SC_BASE_ROW * _COLS

_TC_BLKR = 3072
_TC_ROWS = _SC_BASE_ROW
_TC_GRID = _TC_ROWS // _TC_BLKR


def _make_tc_body(grid_n):
    def _tc_body(src_ref, tar_ref, out_ref, acc_ref):
        i = pl.program_id(0)

        @pl.when(i == 0)
        def _():
            acc_ref[...] = jnp.zeros_like(acc_ref)

        asq = acc_ref[0]
        acn = acc_ref[1]
        for k in range(_TC_BLKR // 8):
            s = src_ref[k * 8:(k + 1) * 8, :]
            t = tar_ref[k * 8:(k + 1) * 8, :]
            mask = t > 0.05
            d = s - t
            asq = asq + jnp.where(mask, d * d, 0.0)
            acn = acn + jnp.where(mask, 1.0, 0.0)
        acc_ref[0] = asq
        acc_ref[1] = acn

        @pl.when(i == grid_n - 1)
        def _():
            out_ref[0] = jnp.sum(acc_ref[0])
            out_ref[1] = jnp.sum(acc_ref[1])
    return _tc_body


def _tc_partial(src2, tar2, row0, grid_n):
    return pl.pallas_call(
        _make_tc_body(grid_n),
        grid=(grid_n,),
        in_specs=[
            pl.BlockSpec((_TC_BLKR, _COLS),
                         lambda i: (row0 // _TC_BLKR + i, 0)),
            pl.BlockSpec((_TC_BLKR, _COLS),
                         lambda i: (row0 // _TC_BLKR + i, 0)),
        ],
        out_specs=pl.BlockSpec(memory_space=pltpu.SMEM),
        out_shape=jax.ShapeDtypeStruct((2,), jnp.float32),
        scratch_shapes=[pltpu.VMEM((2, 8, _COLS), jnp.float32)],
    )(src2, tar2)


def _sc_make():
    mesh = plsc.VectorSubcoreMesh(core_axis_name="c", subcore_axis_name="s")

    @functools.partial(
        pl.kernel,
        mesh=mesh,
        out_type=(
            jax.ShapeDtypeStruct((_SC_NW, 16), jnp.float32),
            jax.ShapeDtypeStruct((_SC_NW, 16), jnp.float32),
        ),
        scratch_types=[
            pltpu.VMEM((_SC_CHUNK_R, _COLS), jnp.float32),
            pltpu.VMEM((_SC_CHUNK_R, _COLS), jnp.float32),
            pltpu.VMEM((_SC_CHUNK_R, _COLS), jnp.float32),
            pltpu.VMEM((_SC_CHUNK_R, _COLS), jnp.float32),
            pltpu.VMEM((16,), jnp.float32),
            pltpu.VMEM((16,), jnp.float32),
            pltpu.SemaphoreType.DMA,
            pltpu.SemaphoreType.DMA,
            pltpu.SemaphoreType.DMA,
            pltpu.SemaphoreType.DMA,
        ],
    )
    def k(src_hbm, tar_hbm, osum, ocnt, sb0, sb1, tb0, tb1, av, cv,
          s0, s1, s2, s3):
        wid = lax.axis_index("s") * 2 + lax.axis_index("c")
        base = _SC_BASE_ROW + wid * (_SC_NCH * _SC_CHUNK_R)
        sbufs = (sb0, sb1)
        tbufs = (tb0, tb1)
        sems = ((s0, s1), (s2, s3))
        copies = {}

        def issue(c):
            b = c % 2
            off = base + c * _SC_CHUNK_R
            copies[c] = (
                pltpu.async_copy(
                    src_hbm.at[pl.ds(off, _SC_CHUNK_R), :], sbufs[b],
                    sems[b][0]),
                pltpu.async_copy(
                    tar_hbm.at[pl.ds(off, _SC_CHUNK_R), :], tbufs[b],
                    sems[b][1]),
            )

        issue(0)
        if _SC_NCH > 1:
            issue(1)

        asq = jnp.zeros((16,), jnp.float32)
        acn = jnp.zeros((16,), jnp.float32)
        for c in range(_SC_NCH):
            b = c % 2
            for cp in copies.pop(c):
                cp.wait()

            def row_body(r, carry, _b=b):
                def body(j, carry2):
                    q, n = carry2
                    s = sbufs[_b][r, pl.ds(j * 16, 16)]
                    t = tbufs[_b][r, pl.ds(j * 16, 16)]
                    m = t > 0.05
                    d = s - t
                    q = q + jnp.where(m, d * d, 0.0)
                    n = n + jnp.where(m, 1.0, 0.0)
                    return (q, n)
                return lax.fori_loop(0, _COLS // 16, body, carry, unroll=8)

            asq, acn = lax.fori_loop(0, _SC_CHUNK_R, row_body, (asq, acn))
            if c + 2 < _SC_NCH:
                issue(c + 2)

        av[...] = asq
        cv[...] = acn
        pltpu.sync_copy(av, osum.at[wid])
        pltpu.sync_copy(cv, ocnt.at[wid])

    return k


def _combine_body(ss_ref, cc_ref, tp_ref, out_ref):
    total = jnp.sum(ss_ref[...]) + tp_ref[0]
    count = jnp.sum(cc_ref[...]) + tp_ref[1]
    out_ref[0] = total / count


def _combine(sc_sums, sc_cnts, tc_part):
    return pl.pallas_call(
        _combine_body,
        in_specs=[
            pl.BlockSpec((_SC_NW, 16), lambda: (0, 0)),
            pl.BlockSpec((_SC_NW, 16), lambda: (0, 0)),
            pl.BlockSpec(memory_space=pltpu.SMEM),
        ],
        out_specs=pl.BlockSpec(memory_space=pltpu.SMEM),
        out_shape=jax.ShapeDtypeStruct((1,), jnp.float32),
    )(sc_sums, sc_cnts, tc_part)


def kernel(src, tar):
    src2 = src.reshape(_ROWS, _COLS)
    tar2 = tar.reshape(_ROWS, _COLS)
    tc_part = _tc_partial(src2, tar2, 0, _TC_GRID)
    sc_sums, sc_cnts = _sc_make()(src2, tar2)
    out = _combine(sc_sums, sc_cnts, tc_part)
    return out[0]


# speedup vs baseline: 1.0506x; 1.0506x over previous
import functools

import jax
import jax.numpy as jnp
from jax import lax
from jax.experimental import pallas as pl
from jax.experimental.pallas import tpu as pltpu
from jax.experimental.pallas import tpu_sc as plsc

_TOT = 32 * 512 * 512
_COLS = 512
_ROWS = _TOT // _COLS

_SC_NW = 16
_SC_CHUNK_R = 32
_SC_NCH = 4
_SC_ROWS = _SC_NW * _SC_NCH * _SC_CHUNK_R
_SC_BASE_ROW = _ROWS - _SC_ROWS
_SC_BASE = _SC_BASE_ROW * _COLS

_TC_BLKR = 2048
_TC_ROWS = _SC_BASE_ROW
_TC_GRID = _TC_ROWS // _TC_BLKR


def _make_tc_body(grid_n):
    def _tc_body(src_ref, tar_ref, out_ref, acc_ref):
        i = pl.program_id(0)

        @pl.when(i == 0)
        def _():
            acc_ref[...] = jnp.zeros_like(acc_ref)

        asq = acc_ref[0]
        acn = acc_ref[1]
        for k in range(_TC_BLKR // 8):
            s = src_ref[k * 8:(k + 1) * 8, :]
            t = tar_ref[k * 8:(k + 1) * 8, :]
            mask = t > 0.05
            d = s - t
            asq = asq + jnp.where(mask, d * d, 0.0)
            acn = acn + jnp.where(mask, 1.0, 0.0)
        acc_ref[0] = asq
        acc_ref[1] = acn

        @pl.when(i == grid_n - 1)
        def _():
            out_ref[0] = jnp.sum(acc_ref[0])
            out_ref[1] = jnp.sum(acc_ref[1])
    return _tc_body


def _tc_partial(src2, tar2, row0, grid_n):
    return pl.pallas_call(
        _make_tc_body(grid_n),
        grid=(grid_n,),
        in_specs=[
            pl.BlockSpec((_TC_BLKR, _COLS),
                         lambda i: (row0 // _TC_BLKR + i, 0)),
            pl.BlockSpec((_TC_BLKR, _COLS),
                         lambda i: (row0 // _TC_BLKR + i, 0)),
        ],
        out_specs=pl.BlockSpec(memory_space=pltpu.SMEM),
        out_shape=jax.ShapeDtypeStruct((2,), jnp.float32),
        scratch_shapes=[pltpu.VMEM((2, 8, _COLS), jnp.float32)],
    )(src2, tar2)


def _sc_make():
    mesh = plsc.VectorSubcoreMesh(core_axis_name="c", subcore_axis_name="s", num_cores=1)

    @functools.partial(
        pl.kernel,
        mesh=mesh,
        out_type=(
            jax.ShapeDtypeStruct((_SC_NW, 16), jnp.float32),
            jax.ShapeDtypeStruct((_SC_NW, 16), jnp.float32),
        ),
        scratch_types=[
            pltpu.VMEM((_SC_CHUNK_R, _COLS), jnp.float32),
            pltpu.VMEM((_SC_CHUNK_R, _COLS), jnp.float32),
            pltpu.VMEM((_SC_CHUNK_R, _COLS), jnp.float32),
            pltpu.VMEM((_SC_CHUNK_R, _COLS), jnp.float32),
            pltpu.VMEM((16,), jnp.float32),
            pltpu.VMEM((16,), jnp.float32),
            pltpu.SemaphoreType.DMA,
            pltpu.SemaphoreType.DMA,
            pltpu.SemaphoreType.DMA,
            pltpu.SemaphoreType.DMA,
        ],
    )
    def k(src_hbm, tar_hbm, osum, ocnt, sb0, sb1, tb0, tb1, av, cv,
          s0, s1, s2, s3):
        wid = lax.axis_index("s")
        base = _SC_BASE_ROW + wid * (_SC_NCH * _SC_CHUNK_R)
        sbufs = (sb0, sb1)
        tbufs = (tb0, tb1)
        sems = ((s0, s1), (s2, s3))
        copies = {}

        def issue(c):
            b = c % 2
            off = base + c * _SC_CHUNK_R
            copies[c] = (
                pltpu.async_copy(
                    src_hbm.at[pl.ds(off, _SC_CHUNK_R), :], sbufs[b],
                    sems[b][0]),
                pltpu.async_copy(
                    tar_hbm.at[pl.ds(off, _SC_CHUNK_R), :], tbufs[b],
                    sems[b][1]),
            )

        issue(0)
        if _SC_NCH > 1:
            issue(1)

        asq = jnp.zeros((16,), jnp.float32)
        acn = jnp.zeros((16,), jnp.float32)
        for c in range(_SC_NCH):
            b = c % 2
            for cp in copies.pop(c):
                cp.wait()

            def row_body(r, carry, _b=b):
                def body(j, carry2):
                    q, n = carry2
                    s = sbufs[_b][r, pl.ds(j * 16, 16)]
                    t = tbufs[_b][r, pl.ds(j * 16, 16)]
                    m = t > 0.05
                    d = s - t
                    q = q + jnp.where(m, d * d, 0.0)
                    n = n + jnp.where(m, 1.0, 0.0)
                    return (q, n)
                return lax.fori_loop(0, _COLS // 16, body, carry, unroll=8)

            asq, acn = lax.fori_loop(0, _SC_CHUNK_R, row_body, (asq, acn))
            if c + 2 < _SC_NCH:
                issue(c + 2)

        av[...] = asq
        cv[...] = acn
        pltpu.sync_copy(av, osum.at[wid])
        pltpu.sync_copy(cv, ocnt.at[wid])

    return k


def _combine_body(ss_ref, cc_ref, tp_ref, out_ref):
    total = jnp.sum(ss_ref[...]) + tp_ref[0]
    count = jnp.sum(cc_ref[...]) + tp_ref[1]
    out_ref[0] = total / count


def _combine(sc_sums, sc_cnts, tc_part):
    return pl.pallas_call(
        _combine_body,
        in_specs=[
            pl.BlockSpec((_SC_NW, 16), lambda: (0, 0)),
            pl.BlockSpec((_SC_NW, 16), lambda: (0, 0)),
            pl.BlockSpec(memory_space=pltpu.SMEM),
        ],
        out_specs=pl.BlockSpec(memory_space=pltpu.SMEM),
        out_shape=jax.ShapeDtypeStruct((1,), jnp.float32),
    )(sc_sums, sc_cnts, tc_part)


def kernel(src, tar):
    src2 = src.reshape(_ROWS, _COLS)
    tar2 = tar.reshape(_ROWS, _COLS)
    tc_part = _tc_partial(src2, tar2, 0, _TC_GRID)
    sc_sums, sc_cnts = _sc_make()(src2, tar2)
    out = _combine(sc_sums, sc_cnts, tc_part)
    return out[0]
